# baseline (speedup 1.0000x reference)
import jax
import jax.numpy as jnp
from jax.experimental import pallas as pl


def _pos_embed_kernel(emb_ref, out_ref):
    t = emb_ref[...].T
    out_ref[...] = jnp.broadcast_to(t[None], out_ref.shape)


def kernel(x, mask, row_embed):
    B = x.shape[0]
    F = x.shape[1]
    L = x.shape[-1]
    LT = 128
    return pl.pallas_call(
        _pos_embed_kernel,
        grid=(L // LT,),
        in_specs=[pl.BlockSpec((LT, F), lambda l: (l, 0))],
        out_specs=pl.BlockSpec((B, F, LT), lambda l: (0, 0, l)),
        out_shape=jax.ShapeDtypeStruct((B, F, L), jnp.float32),
    )(row_embed)


# speedup vs baseline: 1.8200x; 1.8200x over previous
import jax
import jax.numpy as jnp
from jax.experimental import pallas as pl


def _pos_embed_kernel(emb_ref, out_ref):
    t = emb_ref[...].T
    out_ref[...] = jnp.broadcast_to(t[None], out_ref.shape)


def kernel(x, mask, row_embed):
    B = x.shape[0]
    F = x.shape[1]
    L = x.shape[-1]
    LT = 512
    return pl.pallas_call(
        _pos_embed_kernel,
        grid=(L // LT,),
        in_specs=[pl.BlockSpec((LT, F), lambda l: (l, 0))],
        out_specs=pl.BlockSpec((B, F, LT), lambda l: (0, 0, l)),
        out_shape=jax.ShapeDtypeStruct((B, F, L), jnp.float32),
    )(row_embed)
